# baseline (device time: 51692 ns/iter reference)
import jax
import jax.numpy as jnp
from jax import lax
from jax.experimental import pallas as pl
from jax.experimental.pallas import tpu as pltpu

N_DEV = 4

_SLOT = {1: 0, 2: 1, 3: 2}
_N_SCALES = 6


def kernel(x, w_mat):
    m_total, k_shard = x.shape
    k_total, n = w_mat.shape
    m_per = m_total // N_DEV
    m_half = m_per // 2
    m_chunk = m_per // 4

    def body(x_hbm, w_hbm, out_hbm, acc_ref, comm_ref, send_buf, relay_in,
             x_stage, x_loc, w_buf, scale_snd, scale_rcv, send_sems,
             recv_sems, relay_rsems, sc_send_sems, sc_recv_sems,
             stage_sems, x_sem, w_sems, out_sems):
        my = lax.axis_index("i")
        peer = {d: lax.rem(my + d, N_DEV) for d in (1, 2, 3)}

        def stage_half(d, stage_slot, h, sem_idx):
            return pltpu.make_async_copy(
                x_hbm.at[pl.ds(peer[d] * m_per + h * m_half, m_half), :],
                x_stage.at[stage_slot, pl.ds(h * m_half, m_half), :],
                stage_sems.at[sem_idx],
            )

        cps = {
            (1, 0): stage_half(1, 0, 0, 0),
            (1, 1): stage_half(1, 0, 1, 1),
            (3, 0): stage_half(3, 1, 0, 2),
            (3, 1): stage_half(3, 1, 1, 3),
        }
        for cp in cps.values():
            cp.start()

        x_cp = pltpu.make_async_copy(
            x_hbm.at[pl.ds(my * m_per, m_per), :], x_loc, x_sem
        )
        x_cp.start()

        def w_block_copy(src_dev, slot):
            return pltpu.make_async_copy(
                w_hbm.at[pl.ds(src_dev * k_shard, k_shard), :],
                w_buf.at[slot],
                w_sems.at[slot],
            )

        w_cp0 = w_block_copy(my, 0)
        w_cp1 = w_block_copy(lax.rem(my + N_DEV - 1, N_DEV), 1)
        w_cp0.start()
        w_cp1.start()

        barrier_sem = pltpu.get_barrier_semaphore()
        for d in (1, 2, 3):
            pl.semaphore_signal(
                barrier_sem, inc=1,
                device_id=(peer[d],), device_id_type=pl.DeviceIdType.MESH,
            )
        pl.semaphore_wait(barrier_sem, 2)

        def quantize(src_block, slot, row0, rows, scale_slot):
            mx = jnp.maximum(jnp.max(jnp.abs(src_block)), 1e-30)
            send_buf[slot, pl.ds(row0, rows), :] = jnp.round(
                src_block * (127.0 / mx)
            ).astype(jnp.int8)
            scale_snd[scale_slot] = jnp.full((8, 128), mx / 127.0,
                                             jnp.float32)

        def scale_rdma(target, scale_slot):
            return pltpu.make_async_remote_copy(
                src_ref=scale_snd.at[scale_slot],
                dst_ref=scale_rcv.at[scale_slot],
                send_sem=sc_send_sems.at[scale_slot],
                recv_sem=sc_recv_sems.at[scale_slot],
                device_id=(target,),
                device_id_type=pl.DeviceIdType.MESH,
            )

        phase_a = []
        for d, stage_slot, sem_base in ((1, 0, 0), (3, 1, 2)):
            for h in (0, 1):
                cps[(d, h)].wait()
                quantize(
                    x_stage[stage_slot, pl.ds(h * m_half, m_half), :],
                    _SLOT[d], h * m_half, m_half, sem_base + h,
                )
                sc = scale_rdma(peer[d], sem_base + h)
                sc.start()
                da = pltpu.make_async_remote_copy(
                    src_ref=send_buf.at[_SLOT[d],
                                        pl.ds(h * m_half, m_half), :],
                    dst_ref=comm_ref.at[_SLOT[d],
                                        pl.ds(h * m_half, m_half), :],
                    send_sem=send_sems.at[sem_base + h],
                    recv_sem=recv_sems.at[sem_base + h],
                    device_id=(peer[d],),
                    device_id_type=pl.DeviceIdType.MESH,
                )
                da.start()
                phase_a.append((sc, da))

        cp_diag = pltpu.make_async_copy(
            x_hbm.at[pl.ds(peer[2] * m_per, m_per), :],
            x_stage.at[0],
            stage_sems.at[0],
        )
        cp_diag.start()
        cp_diag.wait()
        quantize(x_stage[0, pl.ds(0, m_half), :], _SLOT[2], 0, m_half, 4)
        quantize(x_stage[0, pl.ds(m_half, m_half), :], _SLOT[2], m_half,
                 m_half, 5)

        pl.semaphore_wait(barrier_sem, 1)
        sc_d0 = scale_rdma(peer[2], 4)
        sc_d1 = scale_rdma(peer[2], 5)
        sc_d0.start()
        sc_d1.start()

        def set_strip(row0, src_block, w_slot):
            acc_ref[pl.ds(row0, m_half), :] = jnp.dot(
                src_block, w_buf[w_slot], preferred_element_type=jnp.float32
            )

        def acc_strip(row0, rows, src_block, w_slot, scale_slot):
            acc_ref[pl.ds(row0, rows), :] = acc_ref[
                pl.ds(row0, rows), :
            ] + jnp.dot(
                src_block.astype(jnp.float32), w_buf[w_slot],
                preferred_element_type=jnp.float32,
            ) * scale_rcv[scale_slot, 0:1, 0:1]

        def silu_strip(row0, rows):
            y = acc_ref[pl.ds(row0, rows), :]
            acc_ref[pl.ds(row0, rows), :] = y * jax.nn.sigmoid(y)

        x_cp.wait()
        w_cp0.wait()
        for r in (0, m_half):
            set_strip(r, x_loc[pl.ds(r, m_half), :], 0)
        w_cp0 = w_block_copy(lax.rem(my + 1, N_DEV), 0)
        w_cp0.start()

        def relay_out(k, via_d):
            return pltpu.make_async_remote_copy(
                src_ref=send_buf.at[_SLOT[2], pl.ds(k * m_chunk, m_chunk), :],
                dst_ref=relay_in.at[k],
                send_sem=send_sems.at[4 + k],
                recv_sem=relay_rsems.at[k],
                device_id=(peer[via_d],),
                device_id_type=pl.DeviceIdType.MESH,
            )

        phase_a[0][0].wait_send(); phase_a[0][1].wait_send()
        phase_a[1][0].wait_send(); phase_a[1][1].wait_send()
        ro = [relay_out(0, 1), relay_out(1, 1)]
        ro[0].start()
        ro[1].start()
        phase_a[2][0].wait_send(); phase_a[2][1].wait_send()
        phase_a[3][0].wait_send(); phase_a[3][1].wait_send()
        ro += [relay_out(2, 3), relay_out(3, 3)]
        ro[2].start()
        ro[3].start()

        fwd = []
        for k in (0, 2, 1, 3):
            ro[k].wait_recv()
            f = pltpu.make_async_remote_copy(
                src_ref=relay_in.at[k],
                dst_ref=comm_ref.at[_SLOT[2], pl.ds(k * m_chunk, m_chunk), :],
                send_sem=send_sems.at[8 + k],
                recv_sem=recv_sems.at[4 + k],
                device_id=(peer[1] if k < 2 else peer[3],),
                device_id_type=pl.DeviceIdType.MESH,
            )
            f.start()
            fwd.append(f)

        for h in (0, 1):
            phase_a[h][0].wait_recv()
            phase_a[h][1].wait_recv()
            if h == 0:
                w_cp1.wait()
            acc_strip(h * m_half, m_half,
                      comm_ref[_SLOT[1], pl.ds(h * m_half, m_half), :],
                      1, h)
        w_cp1 = w_block_copy(lax.rem(my + 2, N_DEV), 1)
        w_cp1.start()

        for h in (0, 1):
            phase_a[2 + h][0].wait_recv()
            phase_a[2 + h][1].wait_recv()
            if h == 0:
                w_cp0.wait()
            acc_strip(h * m_half, m_half,
                      comm_ref[_SLOT[3], pl.ds(h * m_half, m_half), :],
                      0, 2 + h)

        w_cp1.wait()
        sc_d0.wait_recv()
        sc_d1.wait_recv()
        out_cps = []
        for c in (0, 2, 1, 3):
            fwd_recv = pltpu.make_async_remote_copy(
                src_ref=comm_ref.at[_SLOT[2], pl.ds(c * m_chunk, m_chunk), :],
                dst_ref=comm_ref.at[_SLOT[2], pl.ds(c * m_chunk, m_chunk), :],
                send_sem=send_sems.at[8 + c],
                recv_sem=recv_sems.at[4 + c],
                device_id=(my,),
                device_id_type=pl.DeviceIdType.MESH,
            )
            fwd_recv.wait_recv()
            r0 = c * m_chunk
            acc_strip(r0, m_chunk,
                      comm_ref[_SLOT[2], pl.ds(r0, m_chunk), :],
                      1, 4 if c < 2 else 5)
            silu_strip(r0, m_chunk)
            ocp = pltpu.make_async_copy(
                acc_ref.at[pl.ds(r0, m_chunk), :],
                out_hbm.at[pl.ds(r0, m_chunk), :],
                out_sems.at[c],
            )
            ocp.start()
            out_cps.append(ocp)

        for k in range(4):
            ro[k].wait_send()
            fwd[k].wait_send()
        sc_d0.wait_send()
        sc_d1.wait_send()
        for ocp in out_cps:
            ocp.wait()

    return pl.pallas_call(
        body,
        out_shape=jax.ShapeDtypeStruct((m_per, n), jnp.float32),
        in_specs=[
            pl.BlockSpec(memory_space=pl.ANY),
            pl.BlockSpec(memory_space=pl.ANY),
        ],
        out_specs=pl.BlockSpec(memory_space=pl.ANY),
        scratch_shapes=[
            pltpu.VMEM((m_per, n), jnp.float32),
            pltpu.VMEM((N_DEV - 1, m_per, k_shard), jnp.int8),
            pltpu.VMEM((N_DEV - 1, m_per, k_shard), jnp.int8),
            pltpu.VMEM((4, m_per // 4, k_shard), jnp.int8),
            pltpu.VMEM((2, m_per, k_shard), jnp.float32),
            pltpu.VMEM((m_per, k_shard), jnp.float32),
            pltpu.VMEM((2, k_shard, n), jnp.float32),
            pltpu.VMEM((_N_SCALES, 8, 128), jnp.float32),
            pltpu.VMEM((_N_SCALES, 8, 128), jnp.float32),
            pltpu.SemaphoreType.DMA((12,)),
            pltpu.SemaphoreType.DMA((8,)),
            pltpu.SemaphoreType.DMA((4,)),
            pltpu.SemaphoreType.DMA((_N_SCALES,)),
            pltpu.SemaphoreType.DMA((_N_SCALES,)),
            pltpu.SemaphoreType.DMA((4,)),
            pltpu.SemaphoreType.DMA,
            pltpu.SemaphoreType.DMA((2,)),
            pltpu.SemaphoreType.DMA((4,)),
        ],
        compiler_params=pltpu.CompilerParams(
            collective_id=0,
            vmem_limit_bytes=60 * 1024 * 1024,
        ),
    )(x, w_mat)


# device time: 42615 ns/iter; 1.2130x vs baseline; 1.2130x over previous
import jax
import jax.numpy as jnp
from jax import lax
from jax.experimental import pallas as pl
from jax.experimental.pallas import tpu as pltpu

N_DEV = 4

_SLOT = {1: 0, 2: 1, 3: 2}
_N_DIAG_CHUNKS = 4
_N_DATA_SEMS = 4 + _N_DIAG_CHUNKS
_N_SCALES = 5


def kernel(x, w_mat):
    m_total, k_shard = x.shape
    k_total, n = w_mat.shape
    m_per = m_total // N_DEV
    m_half = m_per // 2
    m_chunk = m_per // _N_DIAG_CHUNKS

    def body(x_hbm, w_hbm, out_hbm, acc_ref, comm_ref, send_buf, x_stage,
             x_loc, w_buf, scale_snd, scale_rcv, send_sems, recv_sems,
             sc_send_sems, sc_recv_sems, stage_sems, x_sem, w_sems,
             out_sems):
        out_ref = acc_ref
        my = lax.axis_index("i")
        peer = {d: lax.rem(my + d, N_DEV) for d in (1, 2, 3)}

        def stage_half(d, stage_slot, h, sem_idx):
            return pltpu.make_async_copy(
                x_hbm.at[pl.ds(peer[d] * m_per + h * m_half, m_half), :],
                x_stage.at[stage_slot, pl.ds(h * m_half, m_half), :],
                stage_sems.at[sem_idx],
            )

        cps = {
            (1, 0): stage_half(1, 0, 0, 0),
            (1, 1): stage_half(1, 0, 1, 1),
            (3, 0): stage_half(3, 1, 0, 2),
            (3, 1): stage_half(3, 1, 1, 3),
        }
        for cp in cps.values():
            cp.start()

        x_cp = pltpu.make_async_copy(
            x_hbm.at[pl.ds(my * m_per, m_per), :], x_loc, x_sem
        )
        x_cp.start()

        def w_block_copy(src_dev, slot):
            return pltpu.make_async_copy(
                w_hbm.at[pl.ds(src_dev * k_shard, k_shard), :],
                w_buf.at[slot],
                w_sems.at[slot],
            )

        w_cp0 = w_block_copy(my, 0)
        w_cp1 = w_block_copy(lax.rem(my + N_DEV - 1, N_DEV), 1)
        w_cp0.start()
        w_cp1.start()

        barrier_sem = pltpu.get_barrier_semaphore()
        for d in (1, 2, 3):
            pl.semaphore_signal(
                barrier_sem, inc=1,
                device_id=(peer[d],), device_id_type=pl.DeviceIdType.MESH,
            )
        pl.semaphore_wait(barrier_sem, 2)

        def quantize(src_block, slot, row0, rows, scale_slot):
            mx = jnp.maximum(jnp.max(jnp.abs(src_block)), 1e-30)
            send_buf[slot, pl.ds(row0, rows), :] = jnp.round(
                src_block * (127.0 / mx)
            ).astype(jnp.int8)
            scale_snd[scale_slot] = jnp.full((8, 128), mx / 127.0,
                                             jnp.float32)

        def data_rdma(d, row0, rows, sem_idx):
            return pltpu.make_async_remote_copy(
                src_ref=send_buf.at[_SLOT[d], pl.ds(row0, rows), :],
                dst_ref=comm_ref.at[_SLOT[d], pl.ds(row0, rows), :],
                send_sem=send_sems.at[sem_idx],
                recv_sem=recv_sems.at[sem_idx],
                device_id=(peer[d],),
                device_id_type=pl.DeviceIdType.MESH,
            )

        def scale_rdma(d, scale_slot):
            return pltpu.make_async_remote_copy(
                src_ref=scale_snd.at[scale_slot],
                dst_ref=scale_rcv.at[scale_slot],
                send_sem=sc_send_sems.at[scale_slot],
                recv_sem=sc_recv_sems.at[scale_slot],
                device_id=(peer[d],),
                device_id_type=pl.DeviceIdType.MESH,
            )

        phase_a = []
        for d, stage_slot, sem_base in ((1, 0, 0), (3, 1, 2)):
            for h in (0, 1):
                cps[(d, h)].wait()
                quantize(
                    x_stage[stage_slot, pl.ds(h * m_half, m_half), :],
                    _SLOT[d], h * m_half, m_half, sem_base + h,
                )
                sc = scale_rdma(d, sem_base + h)
                sc.start()
                da = data_rdma(d, h * m_half, m_half, sem_base + h)
                da.start()
                phase_a.append((sc, da))

        cp_diag = pltpu.make_async_copy(
            x_hbm.at[pl.ds(peer[2] * m_per, m_per), :],
            x_stage.at[0],
            stage_sems.at[0],
        )
        cp_diag.start()
        cp_diag.wait()
        quantize(x_stage[0], _SLOT[2], 0, m_per, 4)

        def set_strip(row0, src_block, w_slot):
            out_ref[pl.ds(row0, m_half), :] = jnp.dot(
                src_block, w_buf[w_slot], preferred_element_type=jnp.float32
            )

        def acc_strip(row0, rows, src_block, w_slot, scale_slot):
            out_ref[pl.ds(row0, rows), :] = out_ref[
                pl.ds(row0, rows), :
            ] + jnp.dot(
                src_block.astype(jnp.float32), w_buf[w_slot],
                preferred_element_type=jnp.float32,
            ) * scale_rcv[scale_slot, 0:1, 0:1]

        def silu_strip(row0, rows):
            y = out_ref[pl.ds(row0, rows), :]
            out_ref[pl.ds(row0, rows), :] = y * jax.nn.sigmoid(y)

        x_cp.wait()
        w_cp0.wait()
        for r in (0, m_half):
            set_strip(r, x_loc[pl.ds(r, m_half), :], 0)
        w_cp0 = w_block_copy(lax.rem(my + 1, N_DEV), 0)
        w_cp0.start()

        for sc, da in phase_a:
            sc.wait_send()
            da.wait_send()
        pl.semaphore_wait(barrier_sem, 1)
        sc_diag = scale_rdma(2, 4)
        sc_diag.start()
        rdma_d2 = []
        for c in range(_N_DIAG_CHUNKS):
            r = data_rdma(2, c * m_chunk, m_chunk, 4 + c)
            r.start()
            rdma_d2.append(r)

        for h in (0, 1):
            phase_a[h][0].wait_recv()
            phase_a[h][1].wait_recv()
            if h == 0:
                w_cp1.wait()
            acc_strip(h * m_half, m_half,
                      comm_ref[_SLOT[1], pl.ds(h * m_half, m_half), :],
                      1, h)
        w_cp1 = w_block_copy(lax.rem(my + 2, N_DEV), 1)
        w_cp1.start()

        for h in (0, 1):
            phase_a[2 + h][0].wait_recv()
            phase_a[2 + h][1].wait_recv()
            if h == 0:
                w_cp0.wait()
            acc_strip(h * m_half, m_half,
                      comm_ref[_SLOT[3], pl.ds(h * m_half, m_half), :],
                      0, 2 + h)

        w_cp1.wait()
        sc_diag.wait_recv()
        out_cps = []
        for c in range(_N_DIAG_CHUNKS):
            rdma_d2[c].wait_recv()
            r0 = c * m_chunk
            acc_strip(r0, m_chunk,
                      comm_ref[_SLOT[2], pl.ds(r0, m_chunk), :], 1, 4)
            silu_strip(r0, m_chunk)
            ocp = pltpu.make_async_copy(
                acc_ref.at[pl.ds(r0, m_chunk), :],
                out_hbm.at[pl.ds(r0, m_chunk), :],
                out_sems.at[c],
            )
            ocp.start()
            out_cps.append(ocp)

        for c in range(_N_DIAG_CHUNKS):
            rdma_d2[c].wait_send()
        sc_diag.wait_send()
        for ocp in out_cps:
            ocp.wait()

    return pl.pallas_call(
        body,
        out_shape=jax.ShapeDtypeStruct((m_per, n), jnp.float32),
        in_specs=[
            pl.BlockSpec(memory_space=pl.ANY),
            pl.BlockSpec(memory_space=pl.ANY),
        ],
        out_specs=pl.BlockSpec(memory_space=pl.ANY),
        scratch_shapes=[
            pltpu.VMEM((m_per, n), jnp.float32),
            pltpu.VMEM((N_DEV - 1, m_per, k_shard), jnp.int8),
            pltpu.VMEM((N_DEV - 1, m_per, k_shard), jnp.int8),
            pltpu.VMEM((2, m_per, k_shard), jnp.float32),
            pltpu.VMEM((m_per, k_shard), jnp.float32),
            pltpu.VMEM((2, k_shard, n), jnp.float32),
            pltpu.VMEM((_N_SCALES, 8, 128), jnp.float32),
            pltpu.VMEM((_N_SCALES, 8, 128), jnp.float32),
            pltpu.SemaphoreType.DMA((_N_DATA_SEMS,)),
            pltpu.SemaphoreType.DMA((_N_DATA_SEMS,)),
            pltpu.SemaphoreType.DMA((_N_SCALES,)),
            pltpu.SemaphoreType.DMA((_N_SCALES,)),
            pltpu.SemaphoreType.DMA((4,)),
            pltpu.SemaphoreType.DMA,
            pltpu.SemaphoreType.DMA((2,)),
            pltpu.SemaphoreType.DMA((_N_DIAG_CHUNKS,)),
        ],
        compiler_params=pltpu.CompilerParams(
            collective_id=0,
            vmem_limit_bytes=60 * 1024 * 1024,
        ),
    )(x, w_mat)
